# baseline (speedup 1.0000x reference)
import functools

import jax
import jax.numpy as jnp
from jax import lax
from jax.experimental import pallas as pl
from jax.experimental.pallas import tpu as pltpu


def _patches(x_ref, *, K, pad):
    TN, C_in, L = x_ref.shape
    lane = lax.broadcasted_iota(jnp.int32, (C_in, L), 1)
    keeps = {}
    for k in range(K):
        d = k - pad
        if d != 0:
            keeps[k] = (lane < L - d) if d > 0 else (lane >= -d)
    tiles = []
    for n in range(TN):
        xb = x_ref[n].astype(jnp.bfloat16)
        rows = []
        for k in range(K):
            d = k - pad
            if d == 0:
                rows.append(xb)
            else:
                sh = pltpu.roll(xb, (-d) % L, 1)
                rows.append(jnp.where(keeps[k], sh, jnp.bfloat16(0)))
        tiles.append(jnp.concatenate(rows, axis=0))
    return jnp.concatenate(tiles, axis=1) if TN > 1 else tiles[0]


def _gram_body(x_ref, g_ref, u_ref, *, K, pad):
    @pl.when(pl.program_id(1) == 0)
    def _init():
        g_ref[...] = jnp.zeros_like(g_ref)
        u_ref[...] = jnp.zeros_like(u_ref)

    pm = _patches(x_ref, K=K, pad=pad)
    g = lax.dot_general(pm, pm, (((1,), (1,)), ((), ())),
                        preferred_element_type=jnp.float32)
    ones = jnp.ones((pm.shape[1], 128), jnp.bfloat16)
    u = jnp.dot(pm, ones, preferred_element_type=jnp.float32)
    g_ref[0] += g
    u_ref[0] += u


def _apply_body(x_ref, w_ref, g_ref, u_ref, gb_ref, o_ref, *, K, pad, count, eps):
    TN, _, L = x_ref.shape
    w16 = w_ref[...]
    g = jnp.sum(g_ref[...], axis=0)
    u = jnp.sum(u_ref[...], axis=0)[:, 0:1]

    wf32 = w16.astype(jnp.float32)
    h = jnp.dot(w16, g.astype(jnp.bfloat16),
                preferred_element_type=jnp.float32)
    q = jnp.sum(h * wf32, axis=1, keepdims=True)
    s = jnp.dot(w16, u.astype(jnp.bfloat16),
                preferred_element_type=jnp.float32)

    mean = s / count
    var = jnp.maximum(q / count - mean * mean, 0.0)
    gamma = gb_ref[:, 0:1]
    beta = gb_ref[:, 1:2]
    scale = gamma * lax.rsqrt(var + eps)
    shift = beta - mean * scale
    w_bn = (wf32 * scale).astype(jnp.bfloat16)

    pm = _patches(x_ref, K=K, pad=pad)
    conv = jnp.dot(w_bn, pm, preferred_element_type=jnp.float32)
    act = jnp.maximum(conv + shift, 0.0).astype(o_ref.dtype)
    for n in range(TN):
        o_ref[n] = act[:, n * L:(n + 1) * L]


def kernel(x, weight, bias, gamma, beta):
    del bias
    eps = 1e-5
    pad = 1
    N, C_in, L = x.shape
    C_out, _, K = weight.shape
    KC = K * C_in
    assert L + 2 * pad - K + 1 == L, "K=3, pad=1 keeps length"

    wf16 = jnp.transpose(weight, (0, 2, 1)).reshape(C_out, KC).astype(jnp.bfloat16)
    gb = jnp.stack([gamma, beta], axis=1).astype(jnp.float32)

    vmem = 52 * 1024 * 1024

    TS = 32
    while N % TS:
        TS -= 1
    s_tiles = N // TS
    nsplit = 2 if (s_tiles % 2 == 0 and s_tiles >= 2) else 1
    tps = s_tiles // nsplit

    g_acc, u_acc = pl.pallas_call(
        functools.partial(_gram_body, K=K, pad=pad),
        out_shape=(jax.ShapeDtypeStruct((nsplit, KC, KC), jnp.float32),
                   jax.ShapeDtypeStruct((nsplit, KC, 128), jnp.float32)),
        grid=(nsplit, tps),
        in_specs=[
            pl.BlockSpec((TS, C_in, L), lambda s, t: (s * tps + t, 0, 0)),
        ],
        out_specs=(pl.BlockSpec((1, KC, KC), lambda s, t: (s, 0, 0)),
                   pl.BlockSpec((1, KC, 128), lambda s, t: (s, 0, 0))),
        compiler_params=pltpu.CompilerParams(
            dimension_semantics=("parallel", "arbitrary"),
            vmem_limit_bytes=vmem),
    )(x)

    TN = 16
    while N % TN:
        TN -= 1
    n_tiles = N // TN

    out = pl.pallas_call(
        functools.partial(_apply_body, K=K, pad=pad,
                          count=float(N * L), eps=eps),
        out_shape=jax.ShapeDtypeStruct((N, C_out, L), x.dtype),
        grid=(n_tiles,),
        in_specs=[
            pl.BlockSpec((TN, C_in, L), lambda t: (t, 0, 0)),
            pl.BlockSpec((C_out, KC), lambda t: (0, 0)),
            pl.BlockSpec((nsplit, KC, KC), lambda t: (0, 0, 0)),
            pl.BlockSpec((nsplit, KC, 128), lambda t: (0, 0, 0)),
            pl.BlockSpec((C_out, 2), lambda t: (0, 0)),
        ],
        out_specs=pl.BlockSpec((TN, C_out, L), lambda t: (t, 0, 0)),
        compiler_params=pltpu.CompilerParams(
            dimension_semantics=("parallel",),
            vmem_limit_bytes=vmem),
    )(x, wf16, g_acc, u_acc, gb)
    return out


# speedup vs baseline: 1.0482x; 1.0482x over previous
import functools

import jax
import jax.numpy as jnp
from jax import lax
from jax.experimental import pallas as pl
from jax.experimental.pallas import tpu as pltpu


def _patches(x_ref, *, K, pad):
    TN, C_in, L = x_ref.shape
    lane = lax.broadcasted_iota(jnp.int32, (C_in, L), 1)
    keeps = {}
    for k in range(K):
        d = k - pad
        if d != 0:
            keeps[k] = (lane < L - d) if d > 0 else (lane >= -d)
    tiles = []
    for n in range(TN):
        xb = x_ref[n].astype(jnp.bfloat16)
        rows = []
        for k in range(K):
            d = k - pad
            if d == 0:
                rows.append(xb)
            else:
                sh = pltpu.roll(xb, (-d) % L, 1)
                rows.append(jnp.where(keeps[k], sh, jnp.bfloat16(0)))
        tiles.append(jnp.concatenate(rows, axis=0))
    return jnp.concatenate(tiles, axis=1) if TN > 1 else tiles[0]


def _stats_body(x_ref, w_ref, part_ref, *, K, pad):
    pm = _patches(x_ref, K=K, pad=pad)
    conv = jnp.dot(w_ref[...], pm, preferred_element_type=jnp.float32)
    ones = jnp.ones((pm.shape[1], 128), jnp.float32)
    s = jnp.dot(conv, ones, preferred_element_type=jnp.float32)
    q = jnp.dot(conv * conv, ones, preferred_element_type=jnp.float32)
    part_ref[0] = jnp.concatenate([s[:, 0:1], q[:, 0:1]], axis=1)


def _apply_body(x_ref, w_ref, part_ref, gb_ref, o_ref, *, K, pad, count, eps):
    TN, _, L = x_ref.shape
    tot = jnp.sum(part_ref[...], axis=0)
    mean = tot[:, 0:1] / count
    var = jnp.maximum(tot[:, 1:2] / count - mean * mean, 0.0)
    gamma = gb_ref[:, 0:1]
    beta = gb_ref[:, 1:2]
    scale = gamma * lax.rsqrt(var + eps)
    shift = beta - mean * scale
    w_bn = (w_ref[...].astype(jnp.float32) * scale).astype(jnp.bfloat16)

    pm = _patches(x_ref, K=K, pad=pad)
    conv = jnp.dot(w_bn, pm, preferred_element_type=jnp.float32)
    act = jnp.maximum(conv + shift, 0.0).astype(o_ref.dtype)
    for n in range(TN):
        o_ref[n] = act[:, n * L:(n + 1) * L]


def kernel(x, weight, bias, gamma, beta):
    del bias
    eps = 1e-5
    pad = 1
    N, C_in, L = x.shape
    C_out, _, K = weight.shape
    KC = K * C_in
    assert L + 2 * pad - K + 1 == L, "K=3, pad=1 keeps length"

    wf16 = jnp.transpose(weight, (0, 2, 1)).reshape(C_out, KC).astype(jnp.bfloat16)
    gb = jnp.stack([gamma, beta], axis=1).astype(jnp.float32)

    vmem = 52 * 1024 * 1024

    TS = 16
    while N % TS:
        TS -= 1
    s_tiles = N // TS

    parts = pl.pallas_call(
        functools.partial(_stats_body, K=K, pad=pad),
        out_shape=jax.ShapeDtypeStruct((s_tiles, C_out, 2), jnp.float32),
        grid=(s_tiles,),
        in_specs=[
            pl.BlockSpec((TS, C_in, L), lambda t: (t, 0, 0)),
            pl.BlockSpec((C_out, KC), lambda t: (0, 0)),
        ],
        out_specs=pl.BlockSpec((1, C_out, 2), lambda t: (t, 0, 0)),
        compiler_params=pltpu.CompilerParams(
            dimension_semantics=("parallel",),
            vmem_limit_bytes=vmem),
    )(x, wf16)

    TN = 16
    while N % TN:
        TN -= 1
    n_tiles = N // TN

    out = pl.pallas_call(
        functools.partial(_apply_body, K=K, pad=pad,
                          count=float(N * L), eps=eps),
        out_shape=jax.ShapeDtypeStruct((N, C_out, L), x.dtype),
        grid=(n_tiles,),
        in_specs=[
            pl.BlockSpec((TN, C_in, L), lambda t: (t, 0, 0)),
            pl.BlockSpec((C_out, KC), lambda t: (0, 0)),
            pl.BlockSpec((s_tiles, C_out, 2), lambda t: (0, 0, 0)),
            pl.BlockSpec((C_out, 2), lambda t: (0, 0)),
        ],
        out_specs=pl.BlockSpec((TN, C_out, L), lambda t: (t, 0, 0)),
        compiler_params=pltpu.CompilerParams(
            dimension_semantics=("parallel",),
            vmem_limit_bytes=vmem),
    )(x, wf16, parts, gb)
    return out


# speedup vs baseline: 1.3426x; 1.2808x over previous
import functools

import jax
import jax.numpy as jnp
from jax import lax
from jax.experimental import pallas as pl
from jax.experimental.pallas import tpu as pltpu


def _sample_patches(xb, keeps, *, K, pad):
    rows = []
    for k in range(K):
        d = k - pad
        if d == 0:
            rows.append(xb)
        else:
            sh = pltpu.roll(xb, (-d) % xb.shape[1], 1)
            rows.append(jnp.where(keeps[k], sh, jnp.bfloat16(0)))
    return jnp.concatenate(rows, axis=0)


def _tap_keeps(C_in, L, K, pad):
    lane = lax.broadcasted_iota(jnp.int32, (C_in, L), 1)
    keeps = {}
    for k in range(K):
        d = k - pad
        if d != 0:
            keeps[k] = (lane < L - d) if d > 0 else (lane >= -d)
    return keeps


def _stats_body(x_ref, w_ref, acc_ref, *, K, pad):
    TS, C_in, L = x_ref.shape
    C_out = w_ref.shape[0]

    @pl.when(pl.program_id(1) == 0)
    def _init():
        acc_ref[...] = jnp.zeros_like(acc_ref)

    keeps = _tap_keeps(C_in, L, K, pad)
    w = w_ref[...]
    s = jnp.zeros((C_out, 128), jnp.float32)
    q = jnp.zeros((C_out, 128), jnp.float32)
    for n in range(TS):
        pm = _sample_patches(x_ref[n].astype(jnp.bfloat16), keeps, K=K, pad=pad)
        c = jnp.dot(w, pm, preferred_element_type=jnp.float32)
        for j in range(0, L, 128):
            ch = c[:, j:j + 128]
            s = s + ch
            q = q + ch * ch
    acc_ref[0] += jnp.concatenate([s, q], axis=1)


def _apply_body(x_ref, w_ref, acc_ref, gb_ref, o_ref, *, K, pad, count, eps):
    TN, C_in, L = x_ref.shape
    tot = jnp.sum(acc_ref[...], axis=0)
    s = jnp.sum(tot[:, 0:128], axis=1, keepdims=True)
    q = jnp.sum(tot[:, 128:256], axis=1, keepdims=True)
    mean = s / count
    var = jnp.maximum(q / count - mean * mean, 0.0)
    gamma = gb_ref[:, 0:1]
    beta = gb_ref[:, 1:2]
    scale = gamma * lax.rsqrt(var + eps)
    shift = beta - mean * scale
    w_bn = (w_ref[...].astype(jnp.float32) * scale).astype(jnp.bfloat16)

    keeps = _tap_keeps(C_in, L, K, pad)
    tiles = [_sample_patches(x_ref[n].astype(jnp.bfloat16), keeps, K=K, pad=pad)
             for n in range(TN)]
    pm = jnp.concatenate(tiles, axis=1) if TN > 1 else tiles[0]
    conv = jnp.dot(w_bn, pm, preferred_element_type=jnp.float32)
    act = jnp.maximum(conv + shift, 0.0).astype(o_ref.dtype)
    for n in range(TN):
        o_ref[n] = act[:, n * L:(n + 1) * L]


def kernel(x, weight, bias, gamma, beta):
    del bias
    eps = 1e-5
    pad = 1
    N, C_in, L = x.shape
    C_out, _, K = weight.shape
    KC = K * C_in
    assert L + 2 * pad - K + 1 == L, "K=3, pad=1 keeps length"
    assert L % 128 == 0

    wf16 = jnp.transpose(weight, (0, 2, 1)).reshape(C_out, KC).astype(jnp.bfloat16)
    gb = jnp.stack([gamma, beta], axis=1).astype(jnp.float32)

    vmem = 52 * 1024 * 1024

    TS = 16
    while N % TS:
        TS -= 1
    s_tiles = N // TS
    nsplit = 2 if (s_tiles % 2 == 0 and s_tiles >= 2) else 1
    tps = s_tiles // nsplit

    acc = pl.pallas_call(
        functools.partial(_stats_body, K=K, pad=pad),
        out_shape=jax.ShapeDtypeStruct((nsplit, C_out, 256), jnp.float32),
        grid=(nsplit, tps),
        in_specs=[
            pl.BlockSpec((TS, C_in, L), lambda sp, t: (sp * tps + t, 0, 0)),
            pl.BlockSpec((C_out, KC), lambda sp, t: (0, 0)),
        ],
        out_specs=pl.BlockSpec((1, C_out, 256), lambda sp, t: (sp, 0, 0)),
        compiler_params=pltpu.CompilerParams(
            dimension_semantics=("parallel", "arbitrary"),
            vmem_limit_bytes=vmem),
    )(x, wf16)

    TN = 16
    while N % TN:
        TN -= 1
    n_tiles = N // TN

    out = pl.pallas_call(
        functools.partial(_apply_body, K=K, pad=pad,
                          count=float(N * L), eps=eps),
        out_shape=jax.ShapeDtypeStruct((N, C_out, L), x.dtype),
        grid=(n_tiles,),
        in_specs=[
            pl.BlockSpec((TN, C_in, L), lambda t: (t, 0, 0)),
            pl.BlockSpec((C_out, KC), lambda t: (0, 0)),
            pl.BlockSpec((nsplit, C_out, 256), lambda t: (0, 0, 0)),
            pl.BlockSpec((C_out, 2), lambda t: (0, 0)),
        ],
        out_specs=pl.BlockSpec((TN, C_out, L), lambda t: (t, 0, 0)),
        compiler_params=pltpu.CompilerParams(
            dimension_semantics=("parallel",),
            vmem_limit_bytes=vmem),
    )(x, wf16, acc, gb)
    return out
